# Initial kernel scaffold; baseline (speedup 1.0000x reference)
import jax
import jax.numpy as jnp
from jax.experimental import pallas as pl


def kernel(inputs, emb0, emb1, emb2):
    raise NotImplementedError("write your pallas kernel here")


# speedup vs baseline: 1.4344x; 1.4344x over previous
import functools

import jax
import jax.numpy as jnp
from jax import lax
from jax.experimental import pallas as pl
from jax.experimental.pallas import tpu as pltpu
from jax.experimental.pallas import tpu_sc as plsc

D = 384
NC = 2
NS = 16
NW = NC * NS
CH = 128


@functools.lru_cache(maxsize=None)
def _build(B):
    bpw = B // NW
    nch = bpw // CH
    mesh = plsc.VectorSubcoreMesh(core_axis_name="c", subcore_axis_name="s")

    @functools.partial(
        pl.kernel,
        out_type=[jax.ShapeDtypeStruct((B, D), jnp.float32)] * 3,
        mesh=mesh,
        scratch_types=[
            pltpu.VMEM((nch, CH), jnp.int32),
            pltpu.VMEM((bpw, D), jnp.float32),
            pltpu.SemaphoreType.DMA,
        ],
    )
    def emb3(idx_hbm, t0, t1, t2, o0, o1, o2, idx_v, rows_v, sem):
        wid = lax.axis_index("s") * NC + lax.axis_index("c")
        base = wid * bpw
        pltpu.sync_copy(idx_hbm.at[pl.ds(wid * nch, nch)], idx_v)
        for tbl, out in ((t0, o0), (t1, o1), (t2, o2)):
            copies = []
            for j in range(nch):
                copies.append(
                    pltpu.async_copy(
                        tbl.at[idx_v.at[j]],
                        rows_v.at[pl.ds(j * CH, CH)],
                        sem,
                    )
                )
            for c in copies:
                c.wait()
            pltpu.sync_copy(rows_v, out.at[pl.ds(base, bpw)])

    return emb3


def kernel(inputs, emb0, emb1, emb2):
    batch, seq = inputs.shape
    B = batch * seq
    idx = inputs.reshape(B // CH, CH).astype(jnp.int32)
    v0, v1, v2 = _build(B)(idx, emb0, emb1, emb2)
    v0 = v0.reshape(batch, seq, D)
    v1 = v1.reshape(batch, seq, D)
    v2 = v2.reshape(batch, seq, D)
    return (v0, v1, v2, v0, v1, v2)


# speedup vs baseline: 1.4457x; 1.0079x over previous
import functools

import jax
import jax.numpy as jnp
from jax import lax
from jax.experimental import pallas as pl
from jax.experimental.pallas import tpu as pltpu
from jax.experimental.pallas import tpu_sc as plsc

D = 384
NC = 2
NS = 16
NW = NC * NS
CH = 128


@functools.lru_cache(maxsize=None)
def _build(B):
    bpw = B // NW
    nch = bpw // CH
    mesh = plsc.VectorSubcoreMesh(core_axis_name="c", subcore_axis_name="s")

    @functools.partial(
        pl.kernel,
        out_type=[jax.ShapeDtypeStruct((B, D), jnp.float32)] * 3,
        mesh=mesh,
        scratch_types=[
            pltpu.VMEM((nch, CH), jnp.int32),
            pltpu.VMEM((2, CH, D), jnp.float32),
            pltpu.SemaphoreType.DMA,
            pltpu.SemaphoreType.DMA,
            pltpu.SemaphoreType.DMA,
            pltpu.SemaphoreType.DMA,
        ],
    )
    def emb3(idx_hbm, t0, t1, t2, o0, o1, o2, idx_v, rows_v, g0, g1, w0, w1):
        wid = lax.axis_index("s") * NC + lax.axis_index("c")
        base = wid * bpw
        pltpu.sync_copy(idx_hbm.at[pl.ds(wid * nch, nch)], idx_v)
        gsem = (g0, g1)
        wsem = (w0, w1)
        units = [
            (tbl, out, j)
            for tbl, out in ((t0, o0), (t1, o1), (t2, o2))
            for j in range(nch)
        ]
        n = len(units)
        g = [None] * n
        w = [None] * n
        for u, (tbl, out, j) in enumerate(units):
            slot = u % 2
            if u >= 2:
                w[u - 2].wait()
            g[u] = pltpu.async_copy(tbl.at[idx_v.at[j]], rows_v.at[slot], gsem[slot])
            if u >= 1:
                g[u - 1].wait()
                w[u - 1] = pltpu.async_copy(
                    rows_v.at[(u - 1) % 2],
                    units[u - 1][1].at[pl.ds(base + units[u - 1][2] * CH, CH)],
                    wsem[(u - 1) % 2],
                )
        g[n - 1].wait()
        w[n - 1] = pltpu.async_copy(
            rows_v.at[(n - 1) % 2],
            units[n - 1][1].at[pl.ds(base + units[n - 1][2] * CH, CH)],
            wsem[(n - 1) % 2],
        )
        w[n - 2].wait()
        w[n - 1].wait()

    return emb3


def kernel(inputs, emb0, emb1, emb2):
    batch, seq = inputs.shape
    B = batch * seq
    idx = inputs.reshape(B // CH, CH).astype(jnp.int32)
    v0, v1, v2 = _build(B)(idx, emb0, emb1, emb2)
    v0 = v0.reshape(batch, seq, D)
    v1 = v1.reshape(batch, seq, D)
    v2 = v2.reshape(batch, seq, D)
    return (v0, v1, v2, v0, v1, v2)


# speedup vs baseline: 1.8832x; 1.3026x over previous
import functools

import jax
import jax.numpy as jnp
from jax import lax
from jax.experimental import pallas as pl
from jax.experimental.pallas import tpu as pltpu
from jax.experimental.pallas import tpu_sc as plsc

D = 384
NC = 2
NS = 16
NW = NC * NS
CH = 128


@functools.lru_cache(maxsize=None)
def _build(B):
    bpw = B // NW
    nch = bpw // CH
    mesh = plsc.VectorSubcoreMesh(core_axis_name="c", subcore_axis_name="s")

    @functools.partial(
        pl.kernel,
        out_type=[jax.ShapeDtypeStruct((B, D), jnp.float32)] * 6,
        mesh=mesh,
        scratch_types=[
            pltpu.VMEM((nch, CH), jnp.int32),
            pltpu.VMEM((2, CH, D), jnp.float32),
            pltpu.SemaphoreType.DMA,
            pltpu.SemaphoreType.DMA,
            pltpu.SemaphoreType.DMA,
            pltpu.SemaphoreType.DMA,
        ],
    )
    def emb3(idx_hbm, t0, t1, t2, o0, o1, o2, o3, o4, o5, idx_v, rows_v,
             g0, g1, w0, w1):
        wid = lax.axis_index("s") * NC + lax.axis_index("c")
        base = wid * bpw
        pltpu.sync_copy(idx_hbm.at[pl.ds(wid * nch, nch)], idx_v)
        gsem = (g0, g1)
        wsem = (w0, w1)
        units = [
            (tbl, outa, outb, j)
            for tbl, outa, outb in ((t0, o0, o3), (t1, o1, o4), (t2, o2, o5))
            for j in range(nch)
        ]
        n = len(units)
        g = [None] * n
        w = [None] * n

        def fire_writes(u):
            tbl, outa, outb, j = units[u]
            slot = u % 2
            dst = pl.ds(base + j * CH, CH)
            return (
                pltpu.async_copy(rows_v.at[slot], outa.at[dst], wsem[slot]),
                pltpu.async_copy(rows_v.at[slot], outb.at[dst], wsem[slot]),
            )

        for u, (tbl, outa, outb, j) in enumerate(units):
            slot = u % 2
            if u >= 2:
                w[u - 2][0].wait()
                w[u - 2][1].wait()
            g[u] = pltpu.async_copy(tbl.at[idx_v.at[j]], rows_v.at[slot], gsem[slot])
            if u >= 1:
                g[u - 1].wait()
                w[u - 1] = fire_writes(u - 1)
        g[n - 1].wait()
        w[n - 1] = fire_writes(n - 1)
        for d in w[n - 2]:
            d.wait()
        for d in w[n - 1]:
            d.wait()

    return emb3


def kernel(inputs, emb0, emb1, emb2):
    batch, seq = inputs.shape
    B = batch * seq
    idx = inputs.reshape(B // CH, CH).astype(jnp.int32)
    outs = _build(B)(idx, emb0, emb1, emb2)
    return tuple(o.reshape(batch, seq, D) for o in outs)
